# baseline (speedup 1.0000x reference)
import functools

import jax
import jax.numpy as jnp
from jax import lax
from jax.experimental import pallas as pl
from jax.experimental.pallas import tpu as pltpu
from jax.experimental.pallas import tpu_sc as plsc

N = 10000
M = 5000
E = 320000
NC = 2
NS = 16
NW = NC * NS
L = 16
DW = 128
CHUNK = 128
BLK = 8
ZR = 128
M_PAD = 5120
N_PAD = 10112
CPT_E = 80
CPT_F = 160

_MESH = dict(core_axis_name="c", subcore_axis_name="s",
             num_cores=NC, num_subcores=NS)


def _zero_acc(z_h, acc, s, nzc, sem):
    zloops = (nzc + NS - 1) // NS
    for j in range(zloops):
        @pl.when(s + j * NS < nzc)
        def _(j=j):
            pltpu.async_copy(z_h, acc.at[pl.ds((s + j * NS) * ZR, ZR)], sem)
    for j in range(zloops):
        @pl.when(s + j * NS < nzc)
        def _(j=j):
            pltpu.make_async_copy(
                z_h, acc.at[pl.ds((s + j * NS) * ZR, ZR)], sem).wait()


def _write_out(out_h, acc, c, s, nzc, sem):
    zloops = (nzc + NS - 1) // NS
    for j in range(zloops):
        @pl.when(s + j * NS < nzc)
        def _(j=j):
            blk = s + j * NS
            pltpu.async_copy(acc.at[pl.ds(blk * ZR, ZR)],
                             out_h.at[c, pl.ds(blk * ZR, ZR)], sem)
    for j in range(zloops):
        @pl.when(s + j * NS < nzc)
        def _(j=j):
            blk = s + j * NS
            pltpu.make_async_copy(acc.at[pl.ds(blk * ZR, ZR)],
                                  out_h.at[c, pl.ds(blk * ZR, ZR)], sem).wait()


def _gs_loop(tbl, gi_h, si_h, row, gi_v, si_v, dump_v, z_h, buf0, buf1,
             sem0, sem1, sem2, sem3, acc, cpt, ib):
    bufs = (buf0, buf1)
    gsems = (sem0, sem1)
    ssems = (sem2, sem3)

    for b in range(2):
        pltpu.async_copy(bufs[b], acc.at[dump_v], ssems[b], add=True)

    def slab_body(lk, carry):
        pltpu.sync_copy(gi_h.at[row, pl.ds(lk * ib, ib)], gi_v)
        pltpu.sync_copy(si_h.at[row, pl.ds(lk * ib, ib)], si_v)

        def grp_body(g, carry2):
            base = g * 8
            for j in range(8):
                b = j & 1
                pltpu.make_async_copy(z_h, bufs[b], ssems[b]).wait()
                pltpu.async_copy(tbl.at[gi_v.at[base + j]], bufs[b],
                                 gsems[b]).wait()
                pltpu.async_copy(bufs[b], acc.at[si_v.at[base + j]],
                                 ssems[b], add=True)
            return carry2
        lax.fori_loop(0, ib // 8, grp_body, 0)
        for b in range(2):
            pltpu.make_async_copy(z_h, bufs[b], ssems[b]).wait()
            pltpu.async_copy(bufs[b], acc.at[dump_v], ssems[b], add=True)
        return carry
    lax.fori_loop(0, cpt // ib, slab_body, 0)
    for b in range(2):
        pltpu.make_async_copy(z_h, bufs[b], ssems[b]).wait()


def _fill_dump(dump_v, rows_pad):
    val = jnp.full((L,), rows_pad - 1, jnp.int32)
    for g in range(CHUNK // L):
        dump_v[pl.ds(g * L, L)] = val


def _spmm_feat(tbl0, tbl1, gi3, si3, rows_pad, ib):
    nzc = rows_pad // ZR

    @functools.partial(
        pl.kernel,
        out_type=jax.ShapeDtypeStruct((NC, rows_pad, DW), jnp.float32),
        mesh=plsc.VectorSubcoreMesh(**_MESH),
        scratch_types=[
            pltpu.VMEM((ib, CHUNK), jnp.int32),
            pltpu.VMEM((ib, CHUNK), jnp.int32),
            pltpu.VMEM((CHUNK,), jnp.int32),
            pltpu.VMEM((CHUNK, DW), jnp.float32),
            pltpu.VMEM((CHUNK, DW), jnp.float32),
            pltpu.SemaphoreType.DMA,
            pltpu.SemaphoreType.DMA,
            pltpu.SemaphoreType.DMA,
            pltpu.SemaphoreType.DMA,
            pltpu.VMEM_SHARED((rows_pad, DW), jnp.float32),
        ],
    )
    def k(t0_h, t1_h, gi_h, si_h, z_h, out_h, gi_v, si_v, dump_v,
          buf0, buf1, sem0, sem1, sem2, sem3, acc):
        c = lax.axis_index("c")
        s = lax.axis_index("s")
        _fill_dump(dump_v, rows_pad)
        _zero_acc(z_h, acc, s, nzc, sem0)
        plsc.subcore_barrier()

        @pl.when(c == 0)
        def _():
            _gs_loop(t0_h, gi_h, si_h, s, gi_v, si_v, dump_v, z_h,
                     buf0, buf1, sem0, sem1, sem2, sem3, acc, CPT_F, ib)

        @pl.when(c == 1)
        def _():
            _gs_loop(t1_h, gi_h, si_h, s, gi_v, si_v, dump_v, z_h,
                     buf0, buf1, sem0, sem1, sem2, sem3, acc, CPT_F, ib)
        plsc.subcore_barrier()
        _write_out(out_h, acc, c, s, nzc, sem0)

    return k(tbl0, tbl1, gi3, si3, jnp.zeros((ZR, DW), jnp.float32))


def _spmm_edge(tbl, gi3, si3, rows_pad, ib):
    nzc = rows_pad // ZR

    @functools.partial(
        pl.kernel,
        out_type=jax.ShapeDtypeStruct((NC, rows_pad, DW), jnp.float32),
        mesh=plsc.VectorSubcoreMesh(**_MESH),
        scratch_types=[
            pltpu.VMEM((ib, CHUNK), jnp.int32),
            pltpu.VMEM((ib, CHUNK), jnp.int32),
            pltpu.VMEM((CHUNK,), jnp.int32),
            pltpu.VMEM((CHUNK, DW), jnp.float32),
            pltpu.VMEM((CHUNK, DW), jnp.float32),
            pltpu.SemaphoreType.DMA,
            pltpu.SemaphoreType.DMA,
            pltpu.SemaphoreType.DMA,
            pltpu.SemaphoreType.DMA,
            pltpu.VMEM_SHARED((rows_pad, DW), jnp.float32),
        ],
    )
    def k(tbl_h, gi_h, si_h, z_h, out_h, gi_v, si_v, dump_v,
          buf0, buf1, sem0, sem1, sem2, sem3, acc):
        c = lax.axis_index("c")
        s = lax.axis_index("s")
        _fill_dump(dump_v, rows_pad)
        _zero_acc(z_h, acc, s, nzc, sem0)
        plsc.subcore_barrier()
        _gs_loop(tbl_h, gi_h, si_h, c * NS + s, gi_v, si_v, dump_v, z_h,
                 buf0, buf1, sem0, sem1, sem2, sem3, acc, CPT_E, ib)
        plsc.subcore_barrier()
        _write_out(out_h, acc, c, s, nzc, sem0)

    return k(tbl, gi3, si3, jnp.zeros((ZR, DW), jnp.float32))


def _pz_kernel(p, gi3, si3):
    ngr = CHUNK // L

    @functools.partial(
        pl.kernel,
        out_type=jax.ShapeDtypeStruct((NW, M_PAD), jnp.float32),
        mesh=plsc.VectorSubcoreMesh(**_MESH),
        compiler_params=pltpu.CompilerParams(needs_layout_passes=False),
        scratch_types=[
            pltpu.VMEM((N,), jnp.float32),
            pltpu.VMEM((BLK, CHUNK), jnp.int32),
            pltpu.VMEM((BLK, CHUNK), jnp.int32),
            pltpu.VMEM((L * M_PAD,), jnp.float32),
            pltpu.VMEM((M_PAD,), jnp.float32),
        ],
    )
    def k(p_h, gi_h, si_h, out_h, p_v, gi_v, si_v, bank, pz_v):
        c = lax.axis_index("c")
        s = lax.axis_index("s")
        row = c * NS + s
        pltpu.sync_copy(p_h, p_v)
        z16 = jnp.zeros((L,), jnp.float32)

        def zb(i, carry):
            bank[pl.ds(i * L, L)] = z16
            return carry
        lax.fori_loop(0, (L * M_PAD) // L, zb, 0)

        lane_off = lax.iota(jnp.int32, L) * M_PAD

        def blk_body(bk, carry):
            pltpu.sync_copy(gi_h.at[row, pl.ds(bk * BLK, BLK)], gi_v)
            pltpu.sync_copy(si_h.at[row, pl.ds(bk * BLK, BLK)], si_v)

            def body(i, carry2):
                for g in range(ngr):
                    vi = gi_v[i, pl.ds(g * L, L)]
                    vs = si_v[i, pl.ds(g * L, L)]
                    vp = plsc.load_gather(p_v, [vi])
                    plsc.addupdate_scatter(bank, [vs + lane_off], vp)
                return carry2
            lax.fori_loop(0, BLK, body, 0)
            return carry
        lax.fori_loop(0, CPT_E // BLK, blk_body, 0)

        def rbody(mg, carry):
            acc16 = bank[pl.ds(mg * L, L)]
            for lane in range(1, L):
                acc16 = acc16 + bank[pl.ds(lane * M_PAD + mg * L, L)]
            pz_v[pl.ds(mg * L, L)] = acc16
            return carry
        lax.fori_loop(0, M_PAD // L, rbody, 0)
        pltpu.sync_copy(pz_v, out_h.at[row])

    return k(p, gi3, si3)


def _tc_pre(x, W, b2d, an, rW, D):
    nb = D // DW

    def body(*refs):
        x_ref, W_ref, b_ref, an_ref, rW_ref = refs[:5]
        g_refs = refs[5:5 + nb]
        r_ref, p_ref = refs[5 + nb:]
        x_ = x_ref[...]
        h = jnp.dot(x_, W_ref[...], preferred_element_type=jnp.float32)
        h = h + b_ref[...]
        a = jnp.dot(h, an_ref[...], preferred_element_type=jnp.float32)
        p = jnp.exp(a - jnp.max(a))
        g = h * p
        for i, g_ref in enumerate(g_refs):
            g_ref[...] = g[:, i * DW:(i + 1) * DW]
        r_ref[...] = lax.dot_general(x_, rW_ref[...], (((1,), (1,)), ((), ())),
                                     preferred_element_type=jnp.float32)
        p_ref[...] = p

    outs = pl.pallas_call(
        body,
        out_shape=[jax.ShapeDtypeStruct((N, DW), jnp.float32)] * nb
        + [jax.ShapeDtypeStruct((N, D), jnp.float32),
           jax.ShapeDtypeStruct((N, 1), jnp.float32)],
    )(x, W, b2d, an, rW)
    return outs[:nb], outs[nb], outs[nb + 1]


def _tc_mid(S0, S1, pzp, ae, concat):
    def body(S0_ref, S1_ref, pz_ref, ae_ref, u0_ref, u1_ref, z2_ref):
        pz = lax.dot_general(pz_ref[...], jnp.ones((NW, 1), jnp.float32),
                             (((0,), (0,)), ((), ())),
                             preferred_element_type=jnp.float32)[:M]
        Z1 = jnp.sum(pz)
        if concat:
            he0 = S0_ref[...] / Z1
            he1 = S1_ref[...] / Z1
            q = (jnp.dot(he0, ae_ref[:DW],
                         preferred_element_type=jnp.float32)
                 + jnp.dot(he1, ae_ref[DW:],
                           preferred_element_type=jnp.float32))
        else:
            he0 = (S0_ref[...] + S1_ref[...]) / Z1
            he1 = he0
            q = jnp.dot(he0, ae_ref[...], preferred_element_type=jnp.float32)
        qe = jnp.exp(q - jnp.max(q))
        u0_ref[...] = he0 * qe
        if concat:
            u1_ref[...] = he1 * qe
        else:
            u1_ref[...] = jnp.zeros_like(he0)
        z2_ref[...] = jnp.sum(qe * pz).reshape(1, 1)

    return pl.pallas_call(
        body,
        out_shape=[jax.ShapeDtypeStruct((M, DW), jnp.float32),
                   jax.ShapeDtypeStruct((M, DW), jnp.float32),
                   jax.ShapeDtypeStruct((1, 1), jnp.float32)],
    )(S0, S1, pzp, ae)


def _tc_post(t0, t1, p, z2, r, act, D, concat):
    def body(t0_ref, t1_ref, p_ref, z2_ref, r_ref, o_ref):
        if concat:
            t = jnp.concatenate([t0_ref[...], t1_ref[...]], axis=1)
        else:
            t = t0_ref[...] + t1_ref[...]
        o = p_ref[...] * t / z2_ref[...] + r_ref[...]
        if act:
            o = jnp.maximum(o, 0.0)
        o_ref[...] = o

    return pl.pallas_call(
        body,
        out_shape=jax.ShapeDtypeStruct((N, D), jnp.float32),
    )(t0, t1, p, z2, r)


def kernel(x, node_idx, hedge_idx, W1, b1, an1, ae1, rW1, W2, b2, an2, ae2, rW2):
    def padr(a, groups, cpt, val):
        need = groups * cpt * CHUNK - E
        return jnp.concatenate(
            [a, jnp.full((need,), val, a.dtype)]).reshape(groups, cpt, CHUNK)

    ni_e0 = padr(node_idx, NW, CPT_E, 0)
    hi_ed = padr(hedge_idx, NW, CPT_E, M_PAD - 1)
    hi_e0 = padr(hedge_idx, NW, CPT_E, 0)
    ni_ed = padr(node_idx, NW, CPT_E, N_PAD - 1)
    ni_f0 = padr(node_idx, NS, CPT_F, 0)
    hi_fd = padr(hedge_idx, NS, CPT_F, M_PAD - 1)
    hi_f0 = padr(hedge_idx, NS, CPT_F, 0)
    ni_fd = padr(node_idx, NS, CPT_F, N_PAD - 1)

    out = x
    for (W, b, an, ae, rW, D, act) in (
            (W1, b1, an1, ae1, rW1, 256, True),
            (W2, b2, an2, ae2, rW2, 128, False)):
        gs, r, p = _tc_pre(out, W, b.reshape(1, D), an, rW, D)
        pzp = _pz_kernel(p.reshape(N), ni_e0, hi_ed)
        if D == 256:
            S = _spmm_feat(gs[0], gs[1], ni_f0, hi_fd, M_PAD, CPT_F)
            u0, u1, z2 = _tc_mid(S[0, :M], S[1, :M], pzp, ae, True)
            t = _spmm_feat(u0, u1, hi_f0, ni_fd, N_PAD, 16)
            out = _tc_post(t[0, :N], t[1, :N], p, z2, r, act, D, True)
        else:
            S = _spmm_edge(gs[0], ni_e0, hi_ed, M_PAD, CPT_E)
            u0, _u1, z2 = _tc_mid(S[0, :M], S[1, :M], pzp, ae, False)
            t = _spmm_edge(u0, hi_e0, ni_ed, N_PAD, 16)
            out = _tc_post(t[0, :N], t[1, :N], p, z2, r, act, D, False)
    return out


# speedup vs baseline: 1.0235x; 1.0235x over previous
import functools

import jax
import jax.numpy as jnp
from jax import lax
from jax.experimental import pallas as pl
from jax.experimental.pallas import tpu as pltpu
from jax.experimental.pallas import tpu_sc as plsc

N = 10000
M = 5000
E = 320000
NC = 2
NS = 16
NW = NC * NS
L = 16
DW = 128
CHUNK = 128
BLK = 8
ZR = 128
M_PAD = 5120
N_PAD = 10112
CPT_E = 80
CPT_F = 160

_MESH = dict(core_axis_name="c", subcore_axis_name="s",
             num_cores=NC, num_subcores=NS)


def _zero_acc(z_h, acc, s, nzc):
    def zbody(j, carry):
        blk = s + j * NS

        @pl.when(blk < nzc)
        def _():
            pltpu.sync_copy(z_h, acc.at[pl.ds(blk * ZR, ZR)])
        return carry
    lax.fori_loop(0, (nzc + NS - 1) // NS, zbody, 0)


def _write_out(out_h, acc, c, s, nzc):
    def obody(j, carry):
        blk = s + j * NS

        @pl.when(blk < nzc)
        def _():
            pltpu.sync_copy(acc.at[pl.ds(blk * ZR, ZR)],
                            out_h.at[c, pl.ds(blk * ZR, ZR)])
        return carry
    lax.fori_loop(0, (nzc + NS - 1) // NS, obody, 0)


def _gs_loop(tbl, gi_h, si_h, row, gi_v, si_v, buf0, buf1,
             sem0, sem1, sem2, sem3, acc, cpt):
    bufs = (buf0, buf1)
    gsems = (sem0, sem1)
    ssems = (sem2, sem3)

    def blk_body(bk, carry):
        pltpu.sync_copy(gi_h.at[row, pl.ds(bk * BLK, BLK)], gi_v)
        pltpu.sync_copy(si_h.at[row, pl.ds(bk * BLK, BLK)], si_v)
        g_descs = [pltpu.async_copy(tbl.at[gi_v.at[0]], buf0, sem0), None]
        s_descs = [None, None]
        for j in range(BLK):
            b = j & 1
            nb = b ^ 1
            if j + 1 < BLK:
                if s_descs[nb] is not None:
                    s_descs[nb].wait()
                g_descs[nb] = pltpu.async_copy(tbl.at[gi_v.at[j + 1]],
                                               bufs[nb], gsems[nb])
            g_descs[b].wait()
            s_descs[b] = pltpu.async_copy(bufs[b], acc.at[si_v.at[j]],
                                          ssems[b], add=True)
        s_descs[0].wait()
        s_descs[1].wait()
        return carry
    lax.fori_loop(0, cpt // BLK, blk_body, 0)


def _spmm_feat(tbl0, tbl1, gi3, si3, rows_pad):
    nzc = rows_pad // ZR

    @functools.partial(
        pl.kernel,
        out_type=jax.ShapeDtypeStruct((NC, rows_pad, DW), jnp.float32),
        mesh=plsc.VectorSubcoreMesh(**_MESH),
        scratch_types=[
            pltpu.VMEM((BLK, CHUNK), jnp.int32),
            pltpu.VMEM((BLK, CHUNK), jnp.int32),
            pltpu.VMEM((CHUNK, DW), jnp.float32),
            pltpu.VMEM((CHUNK, DW), jnp.float32),
            pltpu.SemaphoreType.DMA,
            pltpu.SemaphoreType.DMA,
            pltpu.SemaphoreType.DMA,
            pltpu.SemaphoreType.DMA,
            pltpu.VMEM_SHARED((rows_pad, DW), jnp.float32),
        ],
    )
    def k(t0_h, t1_h, gi_h, si_h, z_h, out_h, gi_v, si_v, buf0, buf1,
          sem0, sem1, sem2, sem3, acc):
        c = lax.axis_index("c")
        s = lax.axis_index("s")
        _zero_acc(z_h, acc, s, nzc)
        plsc.subcore_barrier()

        @pl.when(c == 0)
        def _():
            _gs_loop(t0_h, gi_h, si_h, s, gi_v, si_v, buf0, buf1,
                     sem0, sem1, sem2, sem3, acc, CPT_F)

        @pl.when(c == 1)
        def _():
            _gs_loop(t1_h, gi_h, si_h, s, gi_v, si_v, buf0, buf1,
                     sem0, sem1, sem2, sem3, acc, CPT_F)
        plsc.subcore_barrier()
        _write_out(out_h, acc, c, s, nzc)

    return k(tbl0, tbl1, gi3, si3, jnp.zeros((ZR, DW), jnp.float32))


def _spmm_edge(tbl, gi3, si3, rows_pad):
    nzc = rows_pad // ZR

    @functools.partial(
        pl.kernel,
        out_type=jax.ShapeDtypeStruct((NC, rows_pad, DW), jnp.float32),
        mesh=plsc.VectorSubcoreMesh(**_MESH),
        scratch_types=[
            pltpu.VMEM((BLK, CHUNK), jnp.int32),
            pltpu.VMEM((BLK, CHUNK), jnp.int32),
            pltpu.VMEM((CHUNK, DW), jnp.float32),
            pltpu.VMEM((CHUNK, DW), jnp.float32),
            pltpu.SemaphoreType.DMA,
            pltpu.SemaphoreType.DMA,
            pltpu.SemaphoreType.DMA,
            pltpu.SemaphoreType.DMA,
            pltpu.VMEM_SHARED((rows_pad, DW), jnp.float32),
        ],
    )
    def k(tbl_h, gi_h, si_h, z_h, out_h, gi_v, si_v, buf0, buf1,
          sem0, sem1, sem2, sem3, acc):
        c = lax.axis_index("c")
        s = lax.axis_index("s")
        _zero_acc(z_h, acc, s, nzc)
        plsc.subcore_barrier()
        _gs_loop(tbl_h, gi_h, si_h, c * NS + s, gi_v, si_v, buf0, buf1,
                 sem0, sem1, sem2, sem3, acc, CPT_E)
        plsc.subcore_barrier()
        _write_out(out_h, acc, c, s, nzc)

    return k(tbl, gi3, si3, jnp.zeros((ZR, DW), jnp.float32))


def _pz_kernel(p, gi3, si3):
    ngr = CHUNK // L

    @functools.partial(
        pl.kernel,
        out_type=jax.ShapeDtypeStruct((NW, M_PAD), jnp.float32),
        mesh=plsc.VectorSubcoreMesh(**_MESH),
        compiler_params=pltpu.CompilerParams(needs_layout_passes=False),
        scratch_types=[
            pltpu.VMEM((N,), jnp.float32),
            pltpu.VMEM((BLK, CHUNK), jnp.int32),
            pltpu.VMEM((BLK, CHUNK), jnp.int32),
            pltpu.VMEM((L * M_PAD,), jnp.float32),
            pltpu.VMEM((M_PAD,), jnp.float32),
        ],
    )
    def k(p_h, gi_h, si_h, out_h, p_v, gi_v, si_v, bank, pz_v):
        c = lax.axis_index("c")
        s = lax.axis_index("s")
        row = c * NS + s
        pltpu.sync_copy(p_h, p_v)
        z16 = jnp.zeros((L,), jnp.float32)

        def zb(i, carry):
            bank[pl.ds(i * L, L)] = z16
            return carry
        lax.fori_loop(0, (L * M_PAD) // L, zb, 0)

        lane_off = lax.iota(jnp.int32, L) * M_PAD

        def blk_body(bk, carry):
            pltpu.sync_copy(gi_h.at[row, pl.ds(bk * BLK, BLK)], gi_v)
            pltpu.sync_copy(si_h.at[row, pl.ds(bk * BLK, BLK)], si_v)

            def body(i, carry2):
                for g in range(ngr):
                    vi = gi_v[i, pl.ds(g * L, L)]
                    vs = si_v[i, pl.ds(g * L, L)]
                    vp = plsc.load_gather(p_v, [vi])
                    plsc.addupdate_scatter(bank, [vs + lane_off], vp)
                return carry2
            lax.fori_loop(0, BLK, body, 0)
            return carry
        lax.fori_loop(0, CPT_E // BLK, blk_body, 0)

        def rbody(mg, carry):
            acc16 = bank[pl.ds(mg * L, L)]
            for lane in range(1, L):
                acc16 = acc16 + bank[pl.ds(lane * M_PAD + mg * L, L)]
            pz_v[pl.ds(mg * L, L)] = acc16
            return carry
        lax.fori_loop(0, M_PAD // L, rbody, 0)
        pltpu.sync_copy(pz_v, out_h.at[row])

    return k(p, gi3, si3)


def _tc_pre(x, W, b2d, an, rW, D):
    nb = D // DW

    def body(*refs):
        x_ref, W_ref, b_ref, an_ref, rW_ref = refs[:5]
        g_refs = refs[5:5 + nb]
        r_ref, p_ref = refs[5 + nb:]
        x_ = x_ref[...]
        h = jnp.dot(x_, W_ref[...], preferred_element_type=jnp.float32)
        h = h + b_ref[...]
        a = jnp.dot(h, an_ref[...], preferred_element_type=jnp.float32)
        p = jnp.exp(a - jnp.max(a))
        g = h * p
        for i, g_ref in enumerate(g_refs):
            g_ref[...] = g[:, i * DW:(i + 1) * DW]
        r_ref[...] = lax.dot_general(x_, rW_ref[...], (((1,), (1,)), ((), ())),
                                     preferred_element_type=jnp.float32)
        p_ref[...] = p

    outs = pl.pallas_call(
        body,
        out_shape=[jax.ShapeDtypeStruct((N, DW), jnp.float32)] * nb
        + [jax.ShapeDtypeStruct((N, D), jnp.float32),
           jax.ShapeDtypeStruct((N, 1), jnp.float32)],
    )(x, W, b2d, an, rW)
    return outs[:nb], outs[nb], outs[nb + 1]


def _tc_mid(S0, S1, pzp, ae, concat):
    def body(S0_ref, S1_ref, pz_ref, ae_ref, u0_ref, u1_ref, z2_ref):
        pz = lax.dot_general(pz_ref[...], jnp.ones((NW, 1), jnp.float32),
                             (((0,), (0,)), ((), ())),
                             preferred_element_type=jnp.float32)[:M]
        Z1 = jnp.sum(pz)
        if concat:
            he0 = S0_ref[...] / Z1
            he1 = S1_ref[...] / Z1
            q = (jnp.dot(he0, ae_ref[:DW],
                         preferred_element_type=jnp.float32)
                 + jnp.dot(he1, ae_ref[DW:],
                           preferred_element_type=jnp.float32))
        else:
            he0 = (S0_ref[...] + S1_ref[...]) / Z1
            he1 = he0
            q = jnp.dot(he0, ae_ref[...], preferred_element_type=jnp.float32)
        qe = jnp.exp(q - jnp.max(q))
        u0_ref[...] = he0 * qe
        if concat:
            u1_ref[...] = he1 * qe
        else:
            u1_ref[...] = jnp.zeros_like(he0)
        z2_ref[...] = jnp.sum(qe * pz).reshape(1, 1)

    return pl.pallas_call(
        body,
        out_shape=[jax.ShapeDtypeStruct((M, DW), jnp.float32),
                   jax.ShapeDtypeStruct((M, DW), jnp.float32),
                   jax.ShapeDtypeStruct((1, 1), jnp.float32)],
    )(S0, S1, pzp, ae)


def _tc_post(t0, t1, p, z2, r, act, D, concat):
    def body(t0_ref, t1_ref, p_ref, z2_ref, r_ref, o_ref):
        if concat:
            t = jnp.concatenate([t0_ref[...], t1_ref[...]], axis=1)
        else:
            t = t0_ref[...] + t1_ref[...]
        o = p_ref[...] * t / z2_ref[...] + r_ref[...]
        if act:
            o = jnp.maximum(o, 0.0)
        o_ref[...] = o

    return pl.pallas_call(
        body,
        out_shape=jax.ShapeDtypeStruct((N, D), jnp.float32),
    )(t0, t1, p, z2, r)


def kernel(x, node_idx, hedge_idx, W1, b1, an1, ae1, rW1, W2, b2, an2, ae2, rW2):
    def padr(a, groups, cpt, val):
        need = groups * cpt * CHUNK - E
        return jnp.concatenate(
            [a, jnp.full((need,), val, a.dtype)]).reshape(groups, cpt, CHUNK)

    ni_e0 = padr(node_idx, NW, CPT_E, 0)
    hi_ed = padr(hedge_idx, NW, CPT_E, M_PAD - 1)
    hi_e0 = padr(hedge_idx, NW, CPT_E, 0)
    ni_ed = padr(node_idx, NW, CPT_E, N_PAD - 1)
    ni_f0 = padr(node_idx, NS, CPT_F, 0)
    hi_fd = padr(hedge_idx, NS, CPT_F, M_PAD - 1)
    hi_f0 = padr(hedge_idx, NS, CPT_F, 0)
    ni_fd = padr(node_idx, NS, CPT_F, N_PAD - 1)

    out = x
    for (W, b, an, ae, rW, D, act) in (
            (W1, b1, an1, ae1, rW1, 256, True),
            (W2, b2, an2, ae2, rW2, 128, False)):
        gs, r, p = _tc_pre(out, W, b.reshape(1, D), an, rW, D)
        pzp = _pz_kernel(p.reshape(N), ni_e0, hi_ed)
        if D == 256:
            S = _spmm_feat(gs[0], gs[1], ni_f0, hi_fd, M_PAD)
            u0, u1, z2 = _tc_mid(S[0, :M], S[1, :M], pzp, ae, True)
            t = _spmm_feat(u0, u1, hi_f0, ni_fd, N_PAD)
            out = _tc_post(t[0, :N], t[1, :N], p, z2, r, act, D, True)
        else:
            S = _spmm_edge(gs[0], ni_e0, hi_ed, M_PAD)
            u0, _u1, z2 = _tc_mid(S[0, :M], S[1, :M], pzp, ae, False)
            t = _spmm_edge(u0, hi_e0, ni_ed, N_PAD)
            out = _tc_post(t[0, :N], t[1, :N], p, z2, r, act, D, False)
    return out
